# baseline (speedup 1.0000x reference)
import functools

import numpy as np
import jax
import jax.numpy as jnp
from jax import lax
from jax.experimental import pallas as pl
from jax.experimental.pallas import tpu as pltpu
from jax.experimental.pallas import tpu_sc as plsc

B, K, D, N = 32, 8192, 128, 256
H = D // 2
TAU = 0.5


def _np_threefry2x32(k0, k1, x0, x1):
    rot = (np.array([13, 15, 26, 6], np.uint32),
           np.array([17, 29, 16, 24], np.uint32))
    ks = (np.uint32(k0), np.uint32(k1),
          np.uint32(np.uint32(k0) ^ np.uint32(k1) ^ np.uint32(0x1BD11BDA)))
    x0 = x0 + ks[0]
    x1 = x1 + ks[1]
    for i in range(5):
        for r in rot[i % 2]:
            x0 = x0 + x1
            x1 = (x1 << r) | (x1 >> np.uint32(32 - r))
            x1 = x0 ^ x1
        x0 = x0 + ks[(i + 1) % 3]
        x1 = x1 + ks[(i + 2) % 3] + np.uint32(i + 1)
    return x0, x1


def _np_random_bits(seed, n):
    x0 = np.zeros(n, dtype=np.uint32)
    x1 = np.arange(n, dtype=np.uint32)
    o0, o1 = _np_threefry2x32(np.uint32(0), np.uint32(seed), x0, x1)
    return o0 ^ o1


def _np_dropout_keep_mask():
    old = np.seterr(over="ignore")
    try:
        bits = _np_random_bits(123, B * K * H)
    finally:
        np.seterr(**old)
    f = ((bits >> np.uint32(9)) | np.uint32(0x3F800000)).view(np.float32)
    u = f - np.float32(1.0)
    return (u < np.float32(0.9)).reshape(B, K, H)


def _np_packed_keep_mask_t():
    kt = _np_dropout_keep_mask().transpose(0, 2, 1)
    arr = kt.reshape(B, 2, 32, K).astype(np.uint32)
    shifts = np.arange(32, dtype=np.uint32).reshape(1, 1, 32, 1)
    return np.ascontiguousarray((arr << shifts).sum(axis=2, dtype=np.uint32))


_KEEP_BITS_T = _np_packed_keep_mask_t()


def _mlp_select_kernel(slots_ref, w1t_ref, b1_ref, w2_ref, b2_ref,
                       keep_ref, logits_ref):
    sT = slots_ref[0]
    hT = jax.lax.dot_general(
        w1t_ref[...], sT, (((1,), (1,)), ((), ())),
        preferred_element_type=jnp.float32)
    hT = jax.nn.relu(hT + b1_ref[...])
    w = keep_ref[0]
    kb = w.shape[1]
    shift = jax.lax.broadcasted_iota(jnp.uint32, (32, kb), 0)
    bits0 = (jnp.broadcast_to(w[0:1, :], (32, kb)) >> shift) & jnp.uint32(1)
    bits1 = (jnp.broadcast_to(w[1:2, :], (32, kb)) >> shift) & jnp.uint32(1)
    keep = jnp.concatenate([bits0, bits1], axis=0) != jnp.uint32(0)
    hT = jnp.where(keep, hT / 0.9, jnp.zeros_like(hT))
    l8 = jax.lax.dot_general(
        w2_ref[...], hT, (((1,), (0,)), ((), ())),
        preferred_element_type=jnp.float32)
    logits_row = l8[0:1, :] + b2_ref[...]
    logits_ref[0] = logits_row


def _sc_select(logits_hbm, g_hbm, probs_hbm, zrow, grow, prow, mscr, iscr):
    wid = lax.axis_index("s") * 2 + lax.axis_index("c")
    pltpu.sync_copy(logits_hbm.at[wid], zrow)
    pltpu.sync_copy(g_hbm.at[wid], grow)
    lane = lax.iota(jnp.int32, 16)

    def scan_body(i, carry):
        maxv, idxv = carry
        v = zrow[pl.ds(i * 16, 16)] + grow[pl.ds(i * 16, 16)]
        upd = v > maxv
        return (jnp.where(upd, v, maxv),
                jnp.where(upd, lane + i * 16, idxv))

    maxv0 = jnp.full((16,), -jnp.inf, jnp.float32)
    idxv0 = jnp.zeros((16,), jnp.int32)
    maxv, idxv = lax.fori_loop(0, K // 16, scan_body, (maxv0, idxv0))
    for step in (8, 4, 2, 1):
        mscr[pl.ds(0, 16)] = maxv
        mscr[pl.ds(16, 16)] = maxv
        iscr[pl.ds(0, 16)] = idxv
        iscr[pl.ds(16, 16)] = idxv
        pm = mscr[pl.ds(step, 16)]
        pi = iscr[pl.ds(step, 16)]
        better = (pm > maxv) | ((pm == maxv) & (pi < idxv))
        maxv = jnp.where(better, pm, maxv)
        idxv = jnp.where(better, pi, idxv)

    def fill_body(i, _):
        prow[pl.ds(i * 16, 16)] = jnp.where(
            lane + i * 16 == idxv, jnp.float32(1.0), jnp.float32(0.0))
        return 0

    lax.fori_loop(0, K // 16, fill_body, 0)
    pltpu.sync_copy(prow, probs_hbm.at[wid])


_sc_select_call = pl.kernel(
    _sc_select,
    out_type=jax.ShapeDtypeStruct((B, K), jnp.float32),
    mesh=plsc.VectorSubcoreMesh(core_axis_name="c", subcore_axis_name="s"),
    scratch_types=[
        pltpu.VMEM((K,), jnp.float32),
        pltpu.VMEM((K,), jnp.float32),
        pltpu.VMEM((K,), jnp.float32),
        pltpu.VMEM((32,), jnp.float32),
        pltpu.VMEM((32,), jnp.int32),
    ],
)


@functools.partial(jax.jit, static_argnums=())
def kernel(slots, attention_weights, W1, b1, W2, b2):
    del attention_weights
    keep_t = jnp.asarray(_KEEP_BITS_T)
    gkey = jax.random.key(456)
    u = jax.random.uniform(gkey, (B, K), minval=1e-6, maxval=1.0 - 1e-6)
    g = -jnp.log(-jnp.log(u))

    w1t = W1.T
    w2row = jnp.zeros((8, H), jnp.float32).at[0].set(W2[:, 0])
    b1col = b1.reshape(H, 1)

    logits = pl.pallas_call(
        _mlp_select_kernel,
        grid=(B, 2),
        in_specs=[
            pl.BlockSpec((1, K // 2, D), lambda b, j: (b, j, 0)),
            pl.BlockSpec((H, D), lambda b, j: (0, 0)),
            pl.BlockSpec((H, 1), lambda b, j: (0, 0)),
            pl.BlockSpec((8, H), lambda b, j: (0, 0)),
            pl.BlockSpec((1,), lambda b, j: (0,)),
            pl.BlockSpec((1, 2, K // 2), lambda b, j: (b, 0, j)),
        ],
        out_specs=pl.BlockSpec((1, 1, K // 2), lambda b, j: (b, 0, j)),
        out_shape=jax.ShapeDtypeStruct((B, 1, K), jnp.float32),
        compiler_params=pltpu.CompilerParams(
            dimension_semantics=("parallel", "parallel"),
        ),
    )(slots, w1t, b1col, w2row, b2, keep_t)
    logits = logits.reshape(B, K)
    probs = _sc_select_call(logits, g)
    return (probs, logits)


# speedup vs baseline: 1.1988x; 1.1988x over previous
import functools

import numpy as np
import jax
import jax.numpy as jnp
from jax import lax
from jax.experimental import pallas as pl
from jax.experimental.pallas import tpu as pltpu
from jax.experimental.pallas import tpu_sc as plsc

B, K, D, N = 32, 8192, 128, 256
H = D // 2
TAU = 0.5


def _np_threefry2x32(k0, k1, x0, x1):
    rot = (np.array([13, 15, 26, 6], np.uint32),
           np.array([17, 29, 16, 24], np.uint32))
    ks = (np.uint32(k0), np.uint32(k1),
          np.uint32(np.uint32(k0) ^ np.uint32(k1) ^ np.uint32(0x1BD11BDA)))
    x0 = x0 + ks[0]
    x1 = x1 + ks[1]
    for i in range(5):
        for r in rot[i % 2]:
            x0 = x0 + x1
            x1 = (x1 << r) | (x1 >> np.uint32(32 - r))
            x1 = x0 ^ x1
        x0 = x0 + ks[(i + 1) % 3]
        x1 = x1 + ks[(i + 2) % 3] + np.uint32(i + 1)
    return x0, x1


def _np_random_bits(seed, n):
    x0 = np.zeros(n, dtype=np.uint32)
    x1 = np.arange(n, dtype=np.uint32)
    o0, o1 = _np_threefry2x32(np.uint32(0), np.uint32(seed), x0, x1)
    return o0 ^ o1


def _np_dropout_keep_mask():
    old = np.seterr(over="ignore")
    try:
        bits = _np_random_bits(123, B * K * H)
    finally:
        np.seterr(**old)
    f = ((bits >> np.uint32(9)) | np.uint32(0x3F800000)).view(np.float32)
    u = f - np.float32(1.0)
    return (u < np.float32(0.9)).reshape(B, K, H)


def _np_packed_keep_mask_t():
    kt = _np_dropout_keep_mask().transpose(0, 2, 1)
    arr = kt.reshape(B, 2, 32, K).astype(np.uint32)
    shifts = np.arange(32, dtype=np.uint32).reshape(1, 1, 32, 1)
    return np.ascontiguousarray((arr << shifts).sum(axis=2, dtype=np.uint32))


_KEEP_BITS_T = _np_packed_keep_mask_t()


def _mlp_select_kernel(slots_ref, w1t_ref, b1_ref, w2_ref, b2_ref,
                       keep_ref, logits_ref):
    sT = slots_ref[0]
    hT = jax.lax.dot_general(
        w1t_ref[...], sT, (((1,), (1,)), ((), ())),
        preferred_element_type=jnp.float32)
    hT = jax.nn.relu(hT + b1_ref[...])
    w = keep_ref[0]
    shift = jax.lax.broadcasted_iota(jnp.uint32, (32, K), 0)
    bits0 = (jnp.broadcast_to(w[0:1, :], (32, K)) >> shift) & jnp.uint32(1)
    bits1 = (jnp.broadcast_to(w[1:2, :], (32, K)) >> shift) & jnp.uint32(1)
    keep = jnp.concatenate([bits0, bits1], axis=0) != jnp.uint32(0)
    hT = jnp.where(keep, hT / 0.9, jnp.zeros_like(hT))
    l8 = jax.lax.dot_general(
        w2_ref[...], hT, (((1,), (0,)), ((), ())),
        preferred_element_type=jnp.float32)
    logits_row = l8[0:1, :] + b2_ref[...]
    logits_ref[0] = logits_row


def _sc_select(logits_hbm, g_hbm, probs_hbm, zrow, grow, prow, mscr, iscr):
    wid = lax.axis_index("s") * 2 + lax.axis_index("c")
    pltpu.sync_copy(logits_hbm.at[wid], zrow)
    pltpu.sync_copy(g_hbm.at[wid], grow)
    lane = lax.iota(jnp.int32, 16)

    def scan_body(i, carry):
        maxv, idxv = carry
        v = zrow[pl.ds(i * 16, 16)] + grow[pl.ds(i * 16, 16)]
        upd = v > maxv
        return (jnp.where(upd, v, maxv),
                jnp.where(upd, lane + i * 16, idxv))

    maxv0 = jnp.full((16,), -jnp.inf, jnp.float32)
    idxv0 = jnp.zeros((16,), jnp.int32)
    maxv, idxv = lax.fori_loop(0, K // 16, scan_body, (maxv0, idxv0))
    for step in (8, 4, 2, 1):
        mscr[pl.ds(0, 16)] = maxv
        mscr[pl.ds(16, 16)] = maxv
        iscr[pl.ds(0, 16)] = idxv
        iscr[pl.ds(16, 16)] = idxv
        pm = mscr[pl.ds(step, 16)]
        pi = iscr[pl.ds(step, 16)]
        better = (pm > maxv) | ((pm == maxv) & (pi < idxv))
        maxv = jnp.where(better, pm, maxv)
        idxv = jnp.where(better, pi, idxv)

    def fill_body(i, _):
        prow[pl.ds(i * 16, 16)] = jnp.where(
            lane + i * 16 == idxv, jnp.float32(1.0), jnp.float32(0.0))
        return 0

    lax.fori_loop(0, K // 16, fill_body, 0)
    pltpu.sync_copy(prow, probs_hbm.at[wid])


_sc_select_call = pl.kernel(
    _sc_select,
    out_type=jax.ShapeDtypeStruct((B, K), jnp.float32),
    mesh=plsc.VectorSubcoreMesh(core_axis_name="c", subcore_axis_name="s"),
    scratch_types=[
        pltpu.VMEM((K,), jnp.float32),
        pltpu.VMEM((K,), jnp.float32),
        pltpu.VMEM((K,), jnp.float32),
        pltpu.VMEM((32,), jnp.float32),
        pltpu.VMEM((32,), jnp.int32),
    ],
)


@functools.partial(jax.jit, static_argnums=())
def kernel(slots, attention_weights, W1, b1, W2, b2):
    del attention_weights
    keep_t = jnp.asarray(_KEEP_BITS_T)
    gkey = jax.random.key(456)
    u = jax.random.uniform(gkey, (B, K), minval=1e-6, maxval=1.0 - 1e-6)
    g = -jnp.log(-jnp.log(u))

    w1t = W1.T
    w2row = jnp.zeros((8, H), jnp.float32).at[0].set(W2[:, 0])
    b1col = b1.reshape(H, 1)

    logits = pl.pallas_call(
        _mlp_select_kernel,
        grid=(B,),
        in_specs=[
            pl.BlockSpec((1, K, D), lambda b: (b, 0, 0)),
            pl.BlockSpec((H, D), lambda b: (0, 0)),
            pl.BlockSpec((H, 1), lambda b: (0, 0)),
            pl.BlockSpec((8, H), lambda b: (0, 0)),
            pl.BlockSpec((1,), lambda b: (0,)),
            pl.BlockSpec((1, 2, K), lambda b: (b, 0, 0)),
        ],
        out_specs=pl.BlockSpec((1, 1, K), lambda b: (b, 0, 0)),
        out_shape=jax.ShapeDtypeStruct((B, 1, K), jnp.float32),
        compiler_params=pltpu.CompilerParams(
            dimension_semantics=("parallel",),
        ),
    )(slots, w1t, b1col, w2row, b2, keep_t)
    logits = logits.reshape(B, K)
    probs = _sc_select_call(logits, g)
    return (probs, logits)
